# baseline (speedup 1.0000x reference)
import jax
import jax.numpy as jnp
from jax.experimental import pallas as pl
from jax.experimental.pallas import tpu as pltpu

_N_RATE = 6
_N_YEAR = 91
_N_GENRE = 25
_N_DIRECTOR = 2186
_EMB = 16
_X2_COLS = 2 + _N_GENRE + _N_DIRECTOR
_TPAD = 128
_BM = 512
_NBUF = 4


def _emb_block(xb, tab_ref):
    rate_idx = xb[:, 0:1]
    year_idx = xb[:, 1:2] + _TPAD
    iota = jax.lax.broadcasted_iota(jnp.int32, (xb.shape[0], 2 * _TPAD), 1)
    oh = jnp.logical_or(iota == rate_idx, iota == year_idx).astype(jnp.bfloat16)
    return jnp.dot(oh, tab_ref[...], preferred_element_type=jnp.float32)


_CSPLIT = 1152


def _tc_body(x2_hbm, wc_ref, tab_ref, out_ref, buf_a, buf_b, sems):
    nblocks = x2_hbm.shape[0] // _BM

    def start_copy(block, slot):
        rows = pl.ds(block * _BM, _BM)
        pltpu.make_async_copy(
            x2_hbm.at[rows, pl.ds(0, _CSPLIT)], buf_a.at[slot], sems.at[slot, 0]
        ).start()
        pltpu.make_async_copy(
            x2_hbm.at[rows, pl.ds(_CSPLIT, _X2_COLS - _CSPLIT)],
            buf_b.at[slot],
            sems.at[slot, 1],
        ).start()

    def wait_copy(block, slot):
        rows = pl.ds(block * _BM, _BM)
        pltpu.make_async_copy(
            x2_hbm.at[rows, pl.ds(0, _CSPLIT)], buf_a.at[slot], sems.at[slot, 0]
        ).wait()
        pltpu.make_async_copy(
            x2_hbm.at[rows, pl.ds(_CSPLIT, _X2_COLS - _CSPLIT)],
            buf_b.at[slot],
            sems.at[slot, 1],
        ).wait()

    for s in range(_NBUF):
        start_copy(s, s)

    def step(i, carry):
        slot = jax.lax.rem(i, _NBUF)
        wait_copy(i, slot)
        xa = buf_a[slot]
        xb = buf_b[slot]
        emb = _emb_block(xa, tab_ref)
        gd = jnp.dot(xa.astype(jnp.bfloat16), wc_ref[:_CSPLIT, :],
                     preferred_element_type=jnp.float32)
        gd = gd + jnp.dot(xb.astype(jnp.bfloat16), wc_ref[_CSPLIT:, :],
                          preferred_element_type=jnp.float32)
        gd = jax.nn.sigmoid(gd)
        out_ref[pl.ds(i * _BM, _BM), :] = jnp.concatenate([emb, gd], axis=1)

        @pl.when(i + _NBUF < nblocks)
        def _():
            start_copy(i + _NBUF, slot)

        return carry

    jax.lax.fori_loop(0, nblocks, step, 0)


def kernel(rate_table, year_table, W_genre, W_director, x2):
    B = x2.shape[0]
    tab = jnp.zeros((2 * _TPAD, 2 * _EMB), jnp.float32)
    tab = tab.at[:_N_RATE, :_EMB].set(rate_table)
    tab = tab.at[_TPAD:_TPAD + _N_YEAR, _EMB:].set(year_table)
    tab = tab.astype(jnp.bfloat16)
    wc = jnp.zeros((_X2_COLS, 2 * _EMB), jnp.float32)
    wc = wc.at[2:2 + _N_GENRE, :_EMB].set(W_genre.T)
    wc = wc.at[2 + _N_GENRE:, _EMB:].set(W_director.T)
    wc = wc.astype(jnp.bfloat16)

    return pl.pallas_call(
        _tc_body,
        in_specs=[
            pl.BlockSpec(memory_space=pl.ANY),
            pl.BlockSpec(memory_space=pltpu.VMEM),
            pl.BlockSpec(memory_space=pltpu.VMEM),
        ],
        out_specs=pl.BlockSpec(memory_space=pltpu.VMEM),
        out_shape=jax.ShapeDtypeStruct((B, 4 * _EMB), jnp.float32),
        scratch_shapes=[
            pltpu.VMEM((_NBUF, _BM, _CSPLIT), jnp.int32),
            pltpu.VMEM((_NBUF, _BM, _X2_COLS - _CSPLIT), jnp.int32),
            pltpu.SemaphoreType.DMA((_NBUF, 2)),
        ],
    )(x2, wc, tab)


# speedup vs baseline: 1.0019x; 1.0019x over previous
import jax
import jax.numpy as jnp
from jax.experimental import pallas as pl
from jax.experimental.pallas import tpu as pltpu

_N_RATE = 6
_N_YEAR = 91
_N_GENRE = 25
_N_DIRECTOR = 2186
_EMB = 16
_X2_COLS = 2 + _N_GENRE + _N_DIRECTOR
_TPAD = 128
_BM = 512
_NBUF = 4


def _emb_block(xb, tab_ref):
    rate_idx = xb[:, 0:1]
    year_idx = xb[:, 1:2] + _TPAD
    iota = jax.lax.broadcasted_iota(jnp.int32, (xb.shape[0], 2 * _TPAD), 1)
    oh = jnp.logical_or(iota == rate_idx, iota == year_idx).astype(jnp.bfloat16)
    return jnp.dot(oh, tab_ref[...], preferred_element_type=jnp.float32)


_CSPLIT = 1152


def _tc_body(x2a_hbm, x2b_hbm, wc_ref, tab_ref, out_ref, buf_a, buf_b, sems):
    nblocks = x2a_hbm.shape[0] // _BM

    def start_copy(block, slot):
        rows = pl.ds(block * _BM, _BM)
        pltpu.make_async_copy(
            x2a_hbm.at[rows, pl.ds(0, _CSPLIT)], buf_a.at[slot], sems.at[slot, 0]
        ).start()
        pltpu.make_async_copy(
            x2b_hbm.at[rows, pl.ds(_CSPLIT, _X2_COLS - _CSPLIT)],
            buf_b.at[slot],
            sems.at[slot, 1],
        ).start()

    def wait_copy(block, slot):
        rows = pl.ds(block * _BM, _BM)
        pltpu.make_async_copy(
            x2a_hbm.at[rows, pl.ds(0, _CSPLIT)], buf_a.at[slot], sems.at[slot, 0]
        ).wait()
        pltpu.make_async_copy(
            x2b_hbm.at[rows, pl.ds(_CSPLIT, _X2_COLS - _CSPLIT)],
            buf_b.at[slot],
            sems.at[slot, 1],
        ).wait()

    for s in range(_NBUF):
        start_copy(s, s)

    def step(i, carry):
        slot = jax.lax.rem(i, _NBUF)
        wait_copy(i, slot)
        xa = buf_a[slot]
        xb = buf_b[slot]
        emb = _emb_block(xa, tab_ref)
        gd = jnp.dot(xa.astype(jnp.bfloat16), wc_ref[:_CSPLIT, :],
                     preferred_element_type=jnp.float32)
        gd = gd + jnp.dot(xb.astype(jnp.bfloat16), wc_ref[_CSPLIT:, :],
                          preferred_element_type=jnp.float32)
        gd = jax.nn.sigmoid(gd)
        out_ref[pl.ds(i * _BM, _BM), :] = jnp.concatenate([emb, gd], axis=1)

        @pl.when(i + _NBUF < nblocks)
        def _():
            start_copy(i + _NBUF, slot)

        return carry

    jax.lax.fori_loop(0, nblocks, step, 0)


def kernel(rate_table, year_table, W_genre, W_director, x2):
    B = x2.shape[0]
    tab = jnp.zeros((2 * _TPAD, 2 * _EMB), jnp.float32)
    tab = tab.at[:_N_RATE, :_EMB].set(rate_table)
    tab = tab.at[_TPAD:_TPAD + _N_YEAR, _EMB:].set(year_table)
    tab = tab.astype(jnp.bfloat16)
    wc = jnp.zeros((_X2_COLS, 2 * _EMB), jnp.float32)
    wc = wc.at[2:2 + _N_GENRE, :_EMB].set(W_genre.T)
    wc = wc.at[2 + _N_GENRE:, _EMB:].set(W_director.T)
    wc = wc.astype(jnp.bfloat16)

    return pl.pallas_call(
        _tc_body,
        in_specs=[
            pl.BlockSpec(memory_space=pl.ANY),
            pl.BlockSpec(memory_space=pl.ANY),
            pl.BlockSpec(memory_space=pltpu.VMEM),
            pl.BlockSpec(memory_space=pltpu.VMEM),
        ],
        out_specs=pl.BlockSpec(memory_space=pltpu.VMEM),
        out_shape=jax.ShapeDtypeStruct((B, 4 * _EMB), jnp.float32),
        scratch_shapes=[
            pltpu.VMEM((_NBUF, _BM, _CSPLIT), jnp.int32),
            pltpu.VMEM((_NBUF, _BM, _X2_COLS - _CSPLIT), jnp.int32),
            pltpu.SemaphoreType.DMA((_NBUF, 2)),
        ],
    )(x2, x2, wc, tab)
